# Initial kernel scaffold; baseline (speedup 1.0000x reference)
import jax
import jax.numpy as jnp
from jax.experimental import pallas as pl


def kernel(objectness, pred_bbox_deltas, anchors):
    raise NotImplementedError("write your pallas kernel here")


# speedup vs baseline: 23.8227x; 23.8227x over previous
import functools

import jax
import jax.numpy as jnp
import numpy as np
from jax.experimental import pallas as pl

N = 20000
B = 4
PRE_NMS_TOP_N = 2000
POST_NMS_TOP_N = 1000
NMS_THRESH = 0.7
SCORE_THRESH = 0.0
MIN_SIZE = 1.0
IMG_H, IMG_W = 800.0, 1333.0
BBOX_XFORM_CLIP = float(np.log(1000.0 / 16.0))


def _decode_kernel(dx_ref, dy_ref, dw_ref, dh_ref,
                   ax1_ref, ay1_ref, ax2_ref, ay2_ref, logit_ref,
                   x1_ref, y1_ref, x2_ref, y2_ref, score_ref):
    ax1 = ax1_ref[...]
    ay1 = ay1_ref[...]
    wa = ax2_ref[...] - ax1
    ha = ay2_ref[...] - ay1
    cxa = ax1 + 0.5 * wa
    cya = ay1 + 0.5 * ha
    dw = jnp.minimum(dw_ref[...], BBOX_XFORM_CLIP)
    dh = jnp.minimum(dh_ref[...], BBOX_XFORM_CLIP)
    pcx = dx_ref[...] * wa + cxa
    pcy = dy_ref[...] * ha + cya
    pw = jnp.exp(dw) * wa
    ph = jnp.exp(dh) * ha
    x1 = jnp.clip(pcx - 0.5 * pw, 0.0, IMG_W)
    y1 = jnp.clip(pcy - 0.5 * ph, 0.0, IMG_H)
    x2 = jnp.clip(pcx + 0.5 * pw, 0.0, IMG_W)
    y2 = jnp.clip(pcy + 0.5 * ph, 0.0, IMG_H)
    scores = jax.nn.sigmoid(logit_ref[...])
    valid = ((x2 - x1) >= MIN_SIZE) & ((y2 - y1) >= MIN_SIZE) & (scores > SCORE_THRESH)
    x1_ref[...] = x1
    y1_ref[...] = y1
    x2_ref[...] = x2
    y2_ref[...] = y2
    score_ref[...] = jnp.where(valid, scores, -1.0)


def _nms_kernel(x1_ref, y1_ref, x2_ref, y2_ref, score_ref, out_ref):
    x1 = x1_ref[...]
    y1 = y1_ref[...]
    x2 = x2_ref[...]
    y2 = y2_ref[...]
    area = (x2 - x1) * (y2 - y1)
    idx = jax.lax.broadcasted_iota(jnp.int32, x1.shape, 1)

    def body(i, keep):
        mi = idx == i
        zero = jnp.float32(0.0)

        def pick(v):
            return jnp.sum(jnp.where(mi, v, zero), axis=1, keepdims=True)

        keep_i = pick(keep)
        x1i, y1i, x2i, y2i, area_i = pick(x1), pick(y1), pick(x2), pick(y2), pick(area)
        w = jnp.clip(jnp.minimum(x2, x2i) - jnp.maximum(x1, x1i), 0.0, None)
        h = jnp.clip(jnp.minimum(y2, y2i) - jnp.maximum(y1, y1i), 0.0, None)
        inter = w * h
        iou = inter / (area + area_i - inter + 1e-9)
        sup = (iou > NMS_THRESH) & (keep_i > 0.5) & (idx > i)
        return jnp.where(sup, zero, keep)

    keep = jax.lax.fori_loop(0, PRE_NMS_TOP_N,
                             body, jnp.ones(x1.shape, jnp.float32))
    out_ref[...] = jnp.where(keep > 0.5, score_ref[...], -1.0)


@jax.jit
def kernel(objectness, pred_bbox_deltas, anchors):
    top_logits, top_idx = jax.lax.top_k(objectness, PRE_NMS_TOP_N)
    d = jnp.take_along_axis(pred_bbox_deltas, top_idx[..., None], axis=1)
    a = anchors[top_idx]

    vec = pl.BlockSpec((B, PRE_NMS_TOP_N), lambda: (0, 0))
    x1, y1, x2, y2, scores = pl.pallas_call(
        _decode_kernel,
        grid=(),
        in_specs=[vec] * 9,
        out_specs=[vec] * 5,
        out_shape=[jax.ShapeDtypeStruct((B, PRE_NMS_TOP_N), jnp.float32)] * 5,
    )(d[..., 0], d[..., 1], d[..., 2], d[..., 3],
      a[..., 0], a[..., 1], a[..., 2], a[..., 3], top_logits)

    order = jnp.argsort(-scores, axis=1)
    x1s = jnp.take_along_axis(x1, order, axis=1)
    y1s = jnp.take_along_axis(y1, order, axis=1)
    x2s = jnp.take_along_axis(x2, order, axis=1)
    y2s = jnp.take_along_axis(y2, order, axis=1)
    ss = jnp.take_along_axis(scores, order, axis=1)

    final_scores = pl.pallas_call(
        _nms_kernel,
        grid=(),
        in_specs=[vec] * 5,
        out_specs=vec,
        out_shape=jax.ShapeDtypeStruct((B, PRE_NMS_TOP_N), jnp.float32),
    )(x1s, y1s, x2s, y2s, ss)

    _, keep_idx = jax.lax.top_k(final_scores, POST_NMS_TOP_N)
    boxes = jnp.stack([x1s, y1s, x2s, y2s], axis=-1)
    return jnp.take_along_axis(boxes, keep_idx[..., None], axis=1)


# speedup vs baseline: 26.2377x; 1.1014x over previous
import functools

import jax
import jax.numpy as jnp
import numpy as np
from jax.experimental import pallas as pl

N = 20000
B = 4
PRE_NMS_TOP_N = 2000
POST_NMS_TOP_N = 1000
NMS_THRESH = 0.7
SCORE_THRESH = 0.0
MIN_SIZE = 1.0
IMG_H, IMG_W = 800.0, 1333.0
BBOX_XFORM_CLIP = float(np.log(1000.0 / 16.0))


def _decode_kernel(dx_ref, dy_ref, dw_ref, dh_ref,
                   ax1_ref, ay1_ref, ax2_ref, ay2_ref, logit_ref,
                   x1_ref, y1_ref, x2_ref, y2_ref, score_ref):
    ax1 = ax1_ref[...]
    ay1 = ay1_ref[...]
    wa = ax2_ref[...] - ax1
    ha = ay2_ref[...] - ay1
    cxa = ax1 + 0.5 * wa
    cya = ay1 + 0.5 * ha
    dw = jnp.minimum(dw_ref[...], BBOX_XFORM_CLIP)
    dh = jnp.minimum(dh_ref[...], BBOX_XFORM_CLIP)
    pcx = dx_ref[...] * wa + cxa
    pcy = dy_ref[...] * ha + cya
    pw = jnp.exp(dw) * wa
    ph = jnp.exp(dh) * ha
    x1 = jnp.clip(pcx - 0.5 * pw, 0.0, IMG_W)
    y1 = jnp.clip(pcy - 0.5 * ph, 0.0, IMG_H)
    x2 = jnp.clip(pcx + 0.5 * pw, 0.0, IMG_W)
    y2 = jnp.clip(pcy + 0.5 * ph, 0.0, IMG_H)
    scores = jax.nn.sigmoid(logit_ref[...])
    valid = ((x2 - x1) >= MIN_SIZE) & ((y2 - y1) >= MIN_SIZE) & (scores > SCORE_THRESH)
    x1_ref[...] = x1
    y1_ref[...] = y1
    x2_ref[...] = x2
    y2_ref[...] = y2
    score_ref[...] = jnp.where(valid, scores, -1.0)


def _nms_kernel(x1_ref, y1_ref, x2_ref, y2_ref, score_ref, out_ref):
    x1 = x1_ref[...]
    y1 = y1_ref[...]
    x2 = x2_ref[...]
    y2 = y2_ref[...]
    area = (x2 - x1) * (y2 - y1)
    idx = jax.lax.broadcasted_iota(jnp.int32, x1.shape, 1)
    packed = jnp.concatenate([x1, y1, x2, y2, area], axis=0)
    idxp = jax.lax.broadcasted_iota(jnp.int32, packed.shape, 1)

    def body(i, keep):
        zero = jnp.float32(0.0)
        p = jnp.sum(jnp.where(idxp == i, packed, zero), axis=1, keepdims=True)
        x1i, y1i, x2i, y2i, area_i = p[0:4], p[4:8], p[8:12], p[12:16], p[16:20]
        keep_i = jnp.sum(jnp.where(idx == i, keep, zero), axis=1, keepdims=True)
        w = jnp.clip(jnp.minimum(x2, x2i) - jnp.maximum(x1, x1i), 0.0, None)
        h = jnp.clip(jnp.minimum(y2, y2i) - jnp.maximum(y1, y1i), 0.0, None)
        inter = w * h
        iou = inter / (area + area_i - inter + 1e-9)
        sup = (iou > NMS_THRESH) & (keep_i > 0.5) & (idx > i)
        return jnp.where(sup, zero, keep)

    keep = jax.lax.fori_loop(0, PRE_NMS_TOP_N,
                             body, jnp.ones(x1.shape, jnp.float32))
    out_ref[...] = jnp.where(keep > 0.5, score_ref[...], -1.0)


@jax.jit
def kernel(objectness, pred_bbox_deltas, anchors):
    top_logits, top_idx = jax.lax.top_k(objectness, PRE_NMS_TOP_N)
    d = jnp.take_along_axis(pred_bbox_deltas, top_idx[..., None], axis=1)
    a = anchors[top_idx]

    vec = pl.BlockSpec((B, PRE_NMS_TOP_N), lambda: (0, 0))
    x1, y1, x2, y2, scores = pl.pallas_call(
        _decode_kernel,
        grid=(),
        in_specs=[vec] * 9,
        out_specs=[vec] * 5,
        out_shape=[jax.ShapeDtypeStruct((B, PRE_NMS_TOP_N), jnp.float32)] * 5,
    )(d[..., 0], d[..., 1], d[..., 2], d[..., 3],
      a[..., 0], a[..., 1], a[..., 2], a[..., 3], top_logits)

    order = jnp.argsort(-scores, axis=1)
    x1s = jnp.take_along_axis(x1, order, axis=1)
    y1s = jnp.take_along_axis(y1, order, axis=1)
    x2s = jnp.take_along_axis(x2, order, axis=1)
    y2s = jnp.take_along_axis(y2, order, axis=1)
    ss = jnp.take_along_axis(scores, order, axis=1)

    final_scores = pl.pallas_call(
        _nms_kernel,
        grid=(),
        in_specs=[vec] * 5,
        out_specs=vec,
        out_shape=jax.ShapeDtypeStruct((B, PRE_NMS_TOP_N), jnp.float32),
    )(x1s, y1s, x2s, y2s, ss)

    _, keep_idx = jax.lax.top_k(final_scores, POST_NMS_TOP_N)
    boxes = jnp.stack([x1s, y1s, x2s, y2s], axis=-1)
    return jnp.take_along_axis(boxes, keep_idx[..., None], axis=1)
